# baseline (speedup 1.0000x reference)
import functools

import jax
import jax.numpy as jnp
from jax import lax
from jax.experimental import pallas as pl
from jax.experimental.pallas import tpu as pltpu
from jax.experimental.pallas import tpu_sc as plsc

_NUM_CORES = 2
_NUM_SUBCORES = 16
_NUM_WORKERS = _NUM_CORES * _NUM_SUBCORES
_COMMON_CHUNKS = ((0, 24), (24, 40), (64, 32))
_LAST_OFF = 96
_LAST_ROWS = (24, 40)
_ROWS_C0 = 120
_ROWS_PAIR = 256
_MAX_CHUNK_ROWS = 40
_NBUF = 3


@functools.partial(jax.jit, static_argnums=(1, 2))
def _position_embedding(table, seq_len, dim):
    assert seq_len == _NUM_SUBCORES * _ROWS_PAIR and dim % 128 == 0
    mesh = plsc.VectorSubcoreMesh(
        core_axis_name="c", subcore_axis_name="s", num_cores=_NUM_CORES
    )

    @functools.partial(
        pl.kernel,
        out_type=jax.ShapeDtypeStruct((seq_len, dim), table.dtype),
        mesh=mesh,
        scratch_types=[
            pltpu.VMEM((_NBUF, _MAX_CHUNK_ROWS, dim), table.dtype),
            pltpu.SemaphoreType.DMA((_NBUF,)),
            pltpu.SemaphoreType.DMA((_NBUF,)),
        ],
    )
    def copy_kernel(table_hbm, out_hbm, buf, in_sems, out_sems):
        cid = lax.axis_index("c")
        sid = lax.axis_index("s")
        base = sid * _ROWS_PAIR + cid * _ROWS_C0

        def fire_in(b, off, rows):
            return pltpu.async_copy(
                table_hbm.at[pl.ds(base + off, rows)],
                buf.at[b, pl.ds(0, rows)],
                in_sems.at[b],
            )

        def fire_out(b, off, rows):
            return pltpu.async_copy(
                buf.at[b, pl.ds(0, rows)],
                out_hbm.at[pl.ds(base + off, rows)],
                out_sems.at[b],
            )

        in_dma = [fire_in(b, off, rows) for b, (off, rows) in enumerate(_COMMON_CHUNKS)]
        out_dma = []
        for b, (off, rows) in enumerate(_COMMON_CHUNKS):
            in_dma[b].wait()
            out_dma.append(fire_out(b, off, rows))
        out_dma[0].wait()

        for core, rows in enumerate(_LAST_ROWS):
            @pl.when(cid == core)
            def _(rows=rows):
                fire_in(0, _LAST_OFF, rows).wait()
                fire_out(0, _LAST_OFF, rows).wait()

        out_dma[1].wait()
        out_dma[2].wait()

    return copy_kernel(table)


def kernel(inputs, table):
    seq_len = inputs.shape[-1]
    return _position_embedding(table, seq_len, table.shape[1])
